# baseline (speedup 1.0000x reference)
import functools

import jax
import jax.numpy as jnp
from jax.experimental import pallas as pl
from jax.experimental.pallas import tpu as pltpu

HIDDEN = 2048
NUM_EXPERTS = 64
BLOCK_M = 2048


def _router_kernel(hs_ref, w_ref, logits_ref, scores_ref, idx_ref):
    hs = hs_ref[...]
    w = w_ref[...]
    logits = jax.lax.dot_general(
        hs, w, (((1,), (1,)), ((), ())), preferred_element_type=jnp.float32
    )
    logits_ref[...] = logits

    iota = jax.lax.broadcasted_iota(jnp.int32, logits.shape, 1)

    m1 = jnp.max(logits, axis=1, keepdims=True)
    i1 = jnp.argmax(logits, axis=1, keepdims=True).astype(jnp.int32)
    masked = jnp.where(iota == i1, -jnp.inf, logits)
    m2 = jnp.max(masked, axis=1, keepdims=True)
    i2 = jnp.argmax(masked, axis=1, keepdims=True).astype(jnp.int32)

    e = jnp.exp(m2 - m1)
    denom = 1.0 + e
    s1 = 1.0 / denom
    s2 = e / denom

    scores_ref[...] = jnp.concatenate([s1, s2], axis=1)
    idx_ref[...] = jnp.concatenate([i1, i2], axis=1)


@functools.partial(jax.jit, static_argnames=())
def _router(hs, weight):
    rows = hs.shape[0]
    grid = (rows // BLOCK_M,)
    return pl.pallas_call(
        _router_kernel,
        grid=grid,
        in_specs=[
            pl.BlockSpec((BLOCK_M, HIDDEN), lambda i: (i, 0)),
            pl.BlockSpec((NUM_EXPERTS, HIDDEN), lambda i: (0, 0)),
        ],
        out_specs=[
            pl.BlockSpec((BLOCK_M, NUM_EXPERTS), lambda i: (i, 0)),
            pl.BlockSpec((BLOCK_M, 2), lambda i: (i, 0)),
            pl.BlockSpec((BLOCK_M, 2), lambda i: (i, 0)),
        ],
        out_shape=[
            jax.ShapeDtypeStruct((rows, NUM_EXPERTS), jnp.float32),
            jax.ShapeDtypeStruct((rows, 2), jnp.float32),
            jax.ShapeDtypeStruct((rows, 2), jnp.int32),
        ],
    )(hs, weight)


def kernel(hidden_states, weight):
    hs = hidden_states.reshape(-1, HIDDEN)
    logits, scores, idx = _router(hs, weight)
    return (logits, scores, idx)
